# Initial kernel scaffold; baseline (speedup 1.0000x reference)
import jax
import jax.numpy as jnp
from jax.experimental import pallas as pl


def kernel(user_feat, src_u2d, dst_u2d, src_d2u, dst_d2u, W1_u2d, b1_u2d, W1_d2u, b1_d2u, W2_u2d, b2_u2d, W2_d2u, b2_d2u):
    raise NotImplementedError("write your pallas kernel here")


# speedup vs baseline: 14.0684x; 14.0684x over previous
import functools

import jax
import jax.numpy as jnp
from jax import lax
from jax.experimental import pallas as pl
from jax.experimental.pallas import tpu as pltpu
from jax.experimental.pallas import tpu_sc as plsc

N = 50000
RP = 50176
E = 800000
EP = 819200
PER_T = EP // 16
CH = 128
NCH = 40
SUP = NCH * CH
NSUP = PER_T // SUP
F = 16
F8 = 8
IN_F = 64
BLK = 1024
NBLK = RP // BLK
TPW = RP // 16
ERC = EP // CH

_mesh = plsc.VectorSubcoreMesh(core_axis_name="c", subcore_axis_name="s")


def _tc1_body(x_ref, w_ref, b_ref, o_ref):
    r = (
        jnp.dot(x_ref[...], w_ref[...], preferred_element_type=jnp.float32)
        + b_ref[...]
    )
    o_ref[0] = r[:, :F8]
    o_ref[1] = r[:, F8:]


def _tc1(uf_p, w1uT, b1u):
    return pl.pallas_call(
        _tc1_body,
        grid=(NBLK,),
        in_specs=[
            pl.BlockSpec((BLK, IN_F), lambda i: (i, 0)),
            pl.BlockSpec((IN_F, F), lambda i: (0, 0)),
            pl.BlockSpec((1, F), lambda i: (0, 0)),
        ],
        out_specs=pl.BlockSpec((2, BLK, F8), lambda i: (0, i, 0)),
        out_shape=jax.ShapeDtypeStruct((2, RP, F8), jnp.float32),
    )(uf_p, w1uT, b1u)


def _sc_stream(table, gidx, widx, rows, idx_g, idx_w, sid, gsem, ssem, acc,
               cnt_idx=None, idx_c=None, ones_v=None, accC=None):
    def body(g, carry):
        row0 = pl.multiple_of(sid * (PER_T // CH) + g * NCH, 8)
        pltpu.sync_copy(gidx.at[pl.ds(row0, NCH), :], idx_g)
        pltpu.sync_copy(widx.at[pl.ds(row0, NCH), :], idx_w)
        if cnt_idx is not None:
            pltpu.sync_copy(cnt_idx.at[pl.ds(row0, NCH), :], idx_c)
        gds = [
            pltpu.async_copy(
                table.at[idx_g.at[k]], rows.at[pl.ds(k * CH, CH), :], gsem
            )
            for k in range(NCH)
        ]
        for d in gds:
            d.wait()
        sds = []
        for k in range(NCH):
            sds.append(pltpu.async_copy(
                rows.at[pl.ds(k * CH, CH), :], acc.at[idx_w.at[k]],
                ssem, add=True))
            if cnt_idx is not None:
                sds.append(pltpu.async_copy(
                    ones_v, accC.at[idx_c.at[k]], ssem, add=True))
        for d in sds:
            d.wait()
        return carry

    lax.fori_loop(0, NSUP, body, 0)


@functools.partial(
    pl.kernel,
    out_type=(
        jax.ShapeDtypeStruct((2, RP, F8), jnp.float32),
        jax.ShapeDtypeStruct((2, RP, F8), jnp.float32),
    ),
    mesh=_mesh,
    compiler_params=pltpu.CompilerParams(use_tc_tiling_on_sc=False),
    scratch_types=[
        pltpu.VMEM((NCH, CH), jnp.int32),
        pltpu.VMEM((NCH, CH), jnp.int32),
        pltpu.VMEM((NCH, CH), jnp.int32),
        pltpu.VMEM((SUP, F8), jnp.float32),
        pltpu.VMEM((CH, F8), jnp.float32),
        pltpu.VMEM_SHARED((RP, F8), jnp.float32),
        pltpu.VMEM_SHARED((RP, F8), jnp.float32),
        pltpu.SemaphoreType.DMA,
        pltpu.SemaphoreType.DMA,
    ],
)
def _sc1(srcU, dstU, dstD, t1h, zeros8, e0rows, outS, outC,
         idx_g, idx_w, idx_c, rows, ones_v, accS, accC, gsem, ssem):
    cid = lax.axis_index("c")
    sid = lax.axis_index("s")
    tz = sid * TPW
    pltpu.sync_copy(zeros8.at[pl.ds(tz, TPW), :], accS.at[pl.ds(tz, TPW), :])
    pltpu.sync_copy(zeros8.at[pl.ds(tz, TPW), :], accC.at[pl.ds(tz, TPW), :])
    pltpu.sync_copy(e0rows, ones_v)
    plsc.subcore_barrier()

    @pl.when(cid == 0)
    def _():
        _sc_stream(t1h.at[0], srcU, dstU, rows, idx_g, idx_w, sid, gsem,
                   ssem, accS, cnt_idx=dstU, idx_c=idx_c, ones_v=ones_v,
                   accC=accC)

    @pl.when(cid == 1)
    def _():
        _sc_stream(t1h.at[1], srcU, dstU, rows, idx_g, idx_w, sid, gsem,
                   ssem, accS, cnt_idx=dstD, idx_c=idx_c, ones_v=ones_v,
                   accC=accC)

    plsc.subcore_barrier()
    pltpu.sync_copy(accS.at[pl.ds(tz, TPW), :], outS.at[cid, pl.ds(tz, TPW), :])
    pltpu.sync_copy(accC.at[pl.ds(tz, TPW), :], outC.at[cid, pl.ds(tz, TPW), :])


def _tc2_body(s_ref, c_ref, b1d_ref, w2dT_ref, b2d_ref, e0_ref,
              hd_ref, hu_ref, t2_ref, fl_ref):
    s = jnp.concatenate([s_ref[0], s_ref[1]], axis=1)
    cntU = c_ref[0][:, 0:1]
    cntD = c_ref[1][:, 0:1]
    hdev = s / jnp.maximum(cntU, 1.0)
    hd = jnp.where(hdev >= 0, hdev, 0.01 * hdev)
    hd_ref[...] = hd
    flag = (cntD > 0).astype(jnp.float32)
    lb = b1d_ref[...]
    lb = jnp.where(lb >= 0, lb, 0.01 * lb)
    hu_ref[...] = flag * lb
    t2_ref[...] = (
        jnp.dot(hd, w2dT_ref[...], preferred_element_type=jnp.float32)
        + b2d_ref[...]
    )
    fl_ref[...] = flag * e0_ref[...]


def _tc2(sp, cp, b1d, w2dT8, b2d8, e0):
    return pl.pallas_call(
        _tc2_body,
        grid=(NBLK,),
        in_specs=[
            pl.BlockSpec((2, BLK, F8), lambda i: (0, i, 0)),
            pl.BlockSpec((2, BLK, F8), lambda i: (0, i, 0)),
            pl.BlockSpec((1, F), lambda i: (0, 0)),
            pl.BlockSpec((F, F8), lambda i: (0, 0)),
            pl.BlockSpec((1, F8), lambda i: (0, 0)),
            pl.BlockSpec((1, F8), lambda i: (0, 0)),
        ],
        out_specs=[
            pl.BlockSpec((BLK, F), lambda i: (i, 0)),
            pl.BlockSpec((BLK, F), lambda i: (i, 0)),
            pl.BlockSpec((BLK, F8), lambda i: (i, 0)),
            pl.BlockSpec((BLK, F8), lambda i: (i, 0)),
        ],
        out_shape=[
            jax.ShapeDtypeStruct((RP, F), jnp.float32),
            jax.ShapeDtypeStruct((RP, F), jnp.float32),
            jax.ShapeDtypeStruct((RP, F8), jnp.float32),
            jax.ShapeDtypeStruct((RP, F8), jnp.float32),
        ],
    )(sp, cp, b1d, w2dT8, b2d8, e0)


@functools.partial(
    pl.kernel,
    out_type=(
        jax.ShapeDtypeStruct((RP, F8), jnp.float32),
        jax.ShapeDtypeStruct((RP, F8), jnp.float32),
    ),
    mesh=_mesh,
    compiler_params=pltpu.CompilerParams(use_tc_tiling_on_sc=False),
    scratch_types=[
        pltpu.VMEM((NCH, CH), jnp.int32),
        pltpu.VMEM((NCH, CH), jnp.int32),
        pltpu.VMEM((SUP, F8), jnp.float32),
        pltpu.VMEM_SHARED((RP, F8), jnp.float32),
        pltpu.VMEM_SHARED((RP, F8), jnp.float32),
        pltpu.SemaphoreType.DMA,
        pltpu.SemaphoreType.DMA,
    ],
)
def _sc2(srcD, dstD, srcU, dstU, t2, flagT, zeros8, outA, outN,
         idx_g, idx_w, rows, accA, accN, gsem, ssem):
    cid = lax.axis_index("c")
    sid = lax.axis_index("s")
    tz = sid * TPW
    pltpu.sync_copy(zeros8.at[pl.ds(tz, TPW), :], accA.at[pl.ds(tz, TPW), :])
    pltpu.sync_copy(zeros8.at[pl.ds(tz, TPW), :], accN.at[pl.ds(tz, TPW), :])
    plsc.subcore_barrier()

    @pl.when(cid == 0)
    def _():
        _sc_stream(t2, srcD, dstD, rows, idx_g, idx_w, sid, gsem, ssem, accA)

    @pl.when(cid == 1)
    def _():
        _sc_stream(flagT, srcU, dstU, rows, idx_g, idx_w, sid, gsem, ssem,
                   accN)

    plsc.subcore_barrier()

    @pl.when(cid == 0)
    def _():
        pltpu.sync_copy(accA.at[pl.ds(tz, TPW), :], outA.at[pl.ds(tz, TPW), :])

    @pl.when(cid == 1)
    def _():
        pltpu.sync_copy(accN.at[pl.ds(tz, TPW), :], outN.at[pl.ds(tz, TPW), :])


def _tc3_body(a2_ref, an_ref, c_ref, b1d_ref, w2uT_ref, b2u_ref,
              o1_ref, o2_ref):
    nA = an_ref[...][:, 0:1]
    cntU = c_ref[0][:, 0:1]
    cntD = c_ref[1][:, 0:1]
    o1_ref[...] = a2_ref[...] / jnp.maximum(cntD, 1.0)
    lb = b1d_ref[...]
    lb = jnp.where(lb >= 0, lb, 0.01 * lb)
    rowA = (
        jnp.dot(lb, w2uT_ref[...], preferred_element_type=jnp.float32)
        + b2u_ref[...]
    )
    rowB = b2u_ref[...]
    frac = nA / jnp.maximum(cntU, 1.0)
    o2_ref[...] = jnp.where(cntU > 0, rowB + frac * (rowA - rowB), 0.0)


def _tc3(a2, an, cp, b1d, w2uT8, b2u8):
    return pl.pallas_call(
        _tc3_body,
        grid=(NBLK,),
        in_specs=[
            pl.BlockSpec((BLK, F8), lambda i: (i, 0)),
            pl.BlockSpec((BLK, F8), lambda i: (i, 0)),
            pl.BlockSpec((2, BLK, F8), lambda i: (0, i, 0)),
            pl.BlockSpec((1, F), lambda i: (0, 0)),
            pl.BlockSpec((F, F8), lambda i: (0, 0)),
            pl.BlockSpec((1, F8), lambda i: (0, 0)),
        ],
        out_specs=[
            pl.BlockSpec((BLK, F8), lambda i: (i, 0)),
            pl.BlockSpec((BLK, F8), lambda i: (i, 0)),
        ],
        out_shape=[
            jax.ShapeDtypeStruct((RP, F8), jnp.float32),
            jax.ShapeDtypeStruct((RP, F8), jnp.float32),
        ],
    )(a2, an, cp, b1d, w2uT8, b2u8)


def kernel(user_feat, src_u2d, dst_u2d, src_d2u, dst_d2u,
           W1_u2d, b1_u2d, W1_d2u, b1_d2u,
           W2_u2d, b2_u2d, W2_d2u, b2_d2u):
    f32 = jnp.float32
    i32 = jnp.int32
    pad_s = jnp.zeros((EP - E,), i32)
    pad_d = jnp.full((EP - E,), RP - 1, i32)

    def prep(src, dst):
        s2 = jnp.concatenate([src.astype(i32), pad_s]).reshape(ERC, CH)
        d2 = jnp.concatenate([dst.astype(i32), pad_d]).reshape(ERC, CH)
        return s2, d2

    srcU2, dstU2 = prep(src_u2d, dst_u2d)
    srcD2, dstD2 = prep(src_d2u, dst_d2u)

    uf_p = jnp.concatenate([user_feat, jnp.zeros((RP - N, IN_F), f32)])
    zeros8 = jnp.zeros((RP, F8), f32)
    e0 = jnp.eye(1, F8, dtype=f32)
    e0rows = jnp.broadcast_to(e0, (CH, F8))

    pad_w = ((0, 0), (0, F8 - 2))
    w1uT = W1_u2d.T
    b1u = b1_u2d.reshape(1, F)
    b1d = b1_d2u.reshape(1, F)
    w2dT8 = jnp.pad(W2_d2u.T, pad_w)
    b2d8 = jnp.pad(b2_d2u.reshape(1, 2), pad_w)
    w2uT8 = jnp.pad(W2_u2d.T, pad_w)
    b2u8 = jnp.pad(b2_u2d.reshape(1, 2), pad_w)

    t1h = _tc1(uf_p, w1uT, b1u)
    sp, cp = _sc1(srcU2, dstU2, dstD2, t1h, zeros8, e0rows)
    hd_p, hu_p, t2, flagT = _tc2(sp, cp, b1d, w2dT8, b2d8, e0)
    a2, an = _sc2(srcD2, dstD2, srcU2, dstU2, t2, flagT, zeros8)
    o1, o2 = _tc3(a2, an, cp, b1d, w2uT8, b2u8)

    h_user2 = o1[:N, :2]
    h_dev2 = o2[:N, :2]
    hu_act = hu_p[:N]
    hd_act = hd_p[:N]
    return (h_user2, h_dev2, hu_act, hd_act)
